# baseline (speedup 1.0000x reference)
import functools
import jax
import jax.numpy as jnp
from jax import lax
from jax.experimental import pallas as pl
from jax.experimental.pallas import tpu as pltpu
from jax.experimental.pallas import tpu_sc as plsc

_info = plsc.get_sparse_core_info()
_NC, _NS, _L = _info.num_cores, _info.num_subcores, _info.num_lanes
_NW = _NC * _NS

_WGROUPS = 2


@functools.lru_cache(maxsize=None)
def _make_gather(B, V, D):
    b_per_w = B // _NW
    n_groups = b_per_w // _L
    mesh = plsc.VectorSubcoreMesh(core_axis_name="c", subcore_axis_name="s")

    @functools.partial(
        pl.kernel,
        out_type=jax.ShapeDtypeStruct((B, D), jnp.float32),
        mesh=mesh,
        compiler_params=pltpu.CompilerParams(needs_layout_passes=False),
        scratch_types=[
            pltpu.VMEM((b_per_w,), jnp.int32),
            pltpu.SemaphoreType.DMA,
        ],
    )
    def gather_kernel(table_hbm, idx_hbm, out_hbm, idx_v, sem):
        wid = lax.axis_index("s") * _NC + lax.axis_index("c")
        base = wid * b_per_w
        pltpu.sync_copy(idx_hbm.at[pl.ds(base, b_per_w)], idx_v)
        lane = lax.iota(jnp.int32, _L)

        def issue_group(g):
            vec = idx_v[pl.ds(pl.multiple_of(g * _L, _L), _L)]
            for j in range(_L):
                s = jnp.sum(jnp.where(lane == j, vec, 0))
                pltpu.async_copy(
                    table_hbm.at[pl.ds(s, 1)],
                    out_hbm.at[pl.ds(base + g * _L + j, 1)],
                    sem,
                )

        def drain_group():
            pltpu.make_async_copy(
                table_hbm.at[pl.ds(0, _L)],
                out_hbm.at[pl.ds(base, _L)],
                sem,
            ).wait()

        def body1(g, carry):
            issue_group(g)
            return carry

        def body2(g, carry):
            issue_group(g)
            drain_group()
            return carry

        def body3(g, carry):
            drain_group()
            return carry

        lax.fori_loop(0, _WGROUPS, body1, 0)
        lax.fori_loop(_WGROUPS, n_groups, body2, 0)
        lax.fori_loop(0, _WGROUPS, body3, 0)

    return gather_kernel


def kernel(tokens, W_E):
    B = tokens.size
    V, D = W_E.shape
    idx = tokens.reshape(B).astype(jnp.int32)
    out = _make_gather(B, V, D)(W_E, idx)
    return out.reshape(*tokens.shape, D)


# speedup vs baseline: 30.3799x; 30.3799x over previous
import functools

import jax
import jax.numpy as jnp
from jax import lax
from jax.experimental import pallas as pl
from jax.experimental.pallas import tpu as pltpu
from jax.experimental.pallas import tpu_sc as plsc

_info = plsc.get_sparse_core_info()
_NC, _NS = _info.num_cores, _info.num_subcores
_NW = _NC * _NS

_CHUNK = 16
_NBUF = 6
_GDEPTH = 3
_WDEPTH = _NBUF - _GDEPTH


@functools.lru_cache(maxsize=None)
def _make_gather(B, V, D):
    assert B % (_NW * _CHUNK) == 0
    b_per_w = B // _NW
    n_chunks = b_per_w // _CHUNK
    mesh = plsc.VectorSubcoreMesh(core_axis_name="c", subcore_axis_name="s")

    @functools.partial(
        pl.kernel,
        out_type=jax.ShapeDtypeStruct((B, D), jnp.float32),
        mesh=mesh,
        scratch_types=[
            pltpu.VMEM((b_per_w,), jnp.int32),
            pltpu.VMEM((_NBUF, _CHUNK, D), jnp.float32),
            pltpu.SemaphoreType.DMA,
        ]
        + [pltpu.SemaphoreType.DMA] * _NBUF,
    )
    def gather_kernel(table_hbm, idx_hbm, out_hbm, idx_v, rows_v, gsem, *wsems):
        wid = lax.axis_index("s") * _NC + lax.axis_index("c")
        base = wid * b_per_w
        pltpu.sync_copy(idx_hbm.at[pl.ds(base, b_per_w)], idx_v)

        def start_gather(g):
            return pltpu.async_copy(
                table_hbm.at[idx_v.at[pl.ds(g * _CHUNK, _CHUNK)]],
                rows_v.at[g % _NBUF],
                gsem,
            )

        gathers = [None] * n_chunks
        writes = [None] * n_chunks
        for g in range(min(_GDEPTH, n_chunks)):
            gathers[g] = start_gather(g)
        for g in range(n_chunks):
            gathers[g].wait()
            if g + _GDEPTH < n_chunks:
                if g - _WDEPTH >= 0:
                    writes[g - _WDEPTH].wait()
                gathers[g + _GDEPTH] = start_gather(g + _GDEPTH)
            writes[g] = pltpu.async_copy(
                rows_v.at[g % _NBUF],
                out_hbm.at[pl.ds(base + g * _CHUNK, _CHUNK)],
                wsems[g % _NBUF],
            )
        for g in range(max(0, n_chunks - _NBUF), n_chunks):
            writes[g].wait()

    return gather_kernel


def kernel(tokens, W_E):
    B = tokens.size
    V, D = W_E.shape
    idx = tokens.reshape(B).astype(jnp.int32)
    out = _make_gather(B, V, D)(W_E, idx)
    return out.reshape(*tokens.shape, D)


# speedup vs baseline: 30.7356x; 1.0117x over previous
import functools

import jax
import jax.numpy as jnp
from jax import lax
from jax.experimental import pallas as pl
from jax.experimental.pallas import tpu as pltpu
from jax.experimental.pallas import tpu_sc as plsc

_info = plsc.get_sparse_core_info()
_NC, _NS = _info.num_cores, _info.num_subcores
_NW = _NC * _NS

_CHUNK = 16
_NBUF = 4
_GDEPTH = 3
_SSLOT = 3


@functools.lru_cache(maxsize=None)
def _make_gather(B, V, D):
    assert B % (_NW * _CHUNK) == 0
    b_per_w = B // _NW
    n_chunks = b_per_w // _CHUNK
    mesh = plsc.VectorSubcoreMesh(core_axis_name="c", subcore_axis_name="s")

    @functools.partial(
        pl.kernel,
        out_type=jax.ShapeDtypeStruct((B, D), jnp.float32),
        mesh=mesh,
        scratch_types=[
            pltpu.VMEM((b_per_w,), jnp.int32),
            pltpu.VMEM((_NBUF, _CHUNK, D), jnp.float32),
            pltpu.VMEM_SHARED((_NS, _SSLOT, _CHUNK, D), jnp.float32),
            pltpu.SemaphoreType.DMA,
        ]
        + [pltpu.SemaphoreType.DMA] * (2 * _SSLOT),
    )
    def gather_kernel(table_hbm, idx_hbm, out_hbm, idx_v, rows_v, sp, gsem, *sems):
        csems, dsems = sems[:_SSLOT], sems[_SSLOT:]
        sid = lax.axis_index("s")
        wid = sid * _NC + lax.axis_index("c")
        base = wid * b_per_w
        pltpu.sync_copy(idx_hbm.at[pl.ds(base, b_per_w)], idx_v)

        def start_gather(g):
            return pltpu.async_copy(
                table_hbm.at[idx_v.at[pl.ds(g * _CHUNK, _CHUNK)]],
                rows_v.at[g % _NBUF],
                gsem,
            )

        def start_stage(g):
            return pltpu.async_copy(
                rows_v.at[g % _NBUF], sp.at[sid, g % _SSLOT], csems[g % _SSLOT]
            )

        def start_write(g):
            return pltpu.async_copy(
                sp.at[sid, g % _SSLOT],
                out_hbm.at[pl.ds(base + g * _CHUNK, _CHUNK)],
                dsems[g % _SSLOT],
            )

        gathers = [None] * n_chunks
        stages = [None] * n_chunks
        writes = [None] * n_chunks
        for g in range(min(_GDEPTH, n_chunks)):
            gathers[g] = start_gather(g)
        for g in range(n_chunks):
            gathers[g].wait()
            if g - _SSLOT >= 0:
                writes[g - _SSLOT].wait()
            stages[g] = start_stage(g)
            if g >= 1:
                stages[g - 1].wait()
                writes[g - 1] = start_write(g - 1)
            if g + _GDEPTH < n_chunks:
                gathers[g + _GDEPTH] = start_gather(g + _GDEPTH)
        stages[n_chunks - 1].wait()
        writes[n_chunks - 1] = start_write(n_chunks - 1)
        for g in range(max(0, n_chunks - _SSLOT), n_chunks):
            writes[g].wait()

    return gather_kernel


def kernel(tokens, W_E):
    B = tokens.size
    V, D = W_E.shape
    idx = tokens.reshape(B).astype(jnp.int32)
    out = _make_gather(B, V, D)(W_E, idx)
    return out.reshape(*tokens.shape, D)
